# Initial kernel scaffold; baseline (speedup 1.0000x reference)
import jax
import jax.numpy as jnp
from jax.experimental import pallas as pl


def kernel(outputs, targets, thin_mask, thin_weight):
    raise NotImplementedError("write your pallas kernel here")


# speedup vs baseline: 2.8832x; 2.8832x over previous
import functools

import jax
import jax.numpy as jnp
from jax import lax
from jax.experimental import pallas as pl
from jax.experimental.pallas import tpu as pltpu
from jax.experimental.pallas import tpu_sc as plsc

_N = 1048576
_NC = 2
_NS = 16
_NW = _NC * _NS
_L = 16
_R = _N // _NW
_CHUNK = 8192
_NCHUNK = _R // _CHUNK
_ITERS = _CHUNK // _L


def _sc_body(o_hbm, t_hbm, m_hbm, out_hbm,
             ob0, ob1, tb0, tb1, mb0, mb1, accv, sem0, sem1):
    wid = lax.axis_index("s") * _NC + lax.axis_index("c")
    iota2 = lax.iota(jnp.int32, _L) * 2

    bufs = ((ob0, tb0, mb0, sem0), (ob1, tb1, mb1, sem1))

    def start_chunk(g, buf):
        ob, tb, mb, sem = buf
        base = wid * _R + g * _CHUNK
        h1 = pltpu.async_copy(o_hbm.at[pl.ds(base * 2, _CHUNK * 2)], ob, sem)
        h2 = pltpu.async_copy(t_hbm.at[pl.ds(base, _CHUNK)], tb, sem)
        h3 = pltpu.async_copy(m_hbm.at[pl.ds(base, _CHUNK)], mb, sem)
        return (h1, h2, h3)

    def do_chunk(buf, acc):
        ob, tb, mb, _ = buf

        def body(j, carry):
            aa, at = carry
            off = j * _L
            t = tb[pl.ds(off, _L)]
            mk = mb[pl.ds(off, _L)]
            idx = iota2 + (j * (2 * _L)) + t
            a = plsc.load_gather(ob, [idx])
            b = plsc.load_gather(ob, [idx ^ 1])
            z = b - a
            mx = jnp.maximum(z, 0.0)
            e = jnp.exp(-jnp.abs(z))
            y = e / (e + 2.0)
            y2 = y * y
            lp = y * (2.0 + y2 * (2.0 / 3.0 + y2 * 0.4))
            sp = mx + lp
            return (aa + sp, at + mk.astype(jnp.float32) * sp)

        return lax.fori_loop(0, _ITERS, body, acc)

    zeros = jnp.zeros((_L,), jnp.float32)
    acc = (zeros, zeros)
    handles = start_chunk(0, bufs[0])
    for g in range(_NCHUNK):
        nxt = start_chunk(g + 1, bufs[(g + 1) % 2]) if g + 1 < _NCHUNK else None
        for h in handles:
            h.wait()
        acc = do_chunk(bufs[g % 2], acc)
        handles = nxt

    accv[pl.ds(0, _L)] = acc[0]
    accv[pl.ds(_L, _L)] = acc[1]
    pltpu.sync_copy(accv, out_hbm.at[pl.ds(wid * (2 * _L), 2 * _L)])


_sc_kernel = functools.partial(
    pl.kernel,
    mesh=plsc.VectorSubcoreMesh(core_axis_name="c", subcore_axis_name="s"),
    out_type=jax.ShapeDtypeStruct((_NW * 2 * _L,), jnp.float32),
    scratch_types=[
        pltpu.VMEM((_CHUNK * 2,), jnp.float32),
        pltpu.VMEM((_CHUNK * 2,), jnp.float32),
        pltpu.VMEM((_CHUNK,), jnp.int32),
        pltpu.VMEM((_CHUNK,), jnp.int32),
        pltpu.VMEM((_CHUNK,), jnp.int32),
        pltpu.VMEM((_CHUNK,), jnp.int32),
        pltpu.VMEM((2 * _L,), jnp.float32),
        pltpu.SemaphoreType.DMA,
        pltpu.SemaphoreType.DMA,
    ],
    compiler_params=pltpu.CompilerParams(needs_layout_passes=False),
)(_sc_body)


def kernel(outputs, targets, thin_mask, thin_weight):
    o_flat = outputs.reshape(-1)
    partials = _sc_kernel(o_flat, targets, thin_mask)
    pr = partials.reshape(_NW, 2, _L)
    s_all = jnp.sum(pr[:, 0, :])
    s_thin = jnp.sum(pr[:, 1, :])
    tw = jnp.asarray(thin_weight, jnp.float32)
    loss = (s_all + (tw - 1.0) * s_thin) * (1.0 / _N)
    return loss.astype(jnp.float32)
